# Initial kernel scaffold; baseline (speedup 1.0000x reference)
import jax
import jax.numpy as jnp
from jax.experimental import pallas as pl


def kernel(v, e, vidx, eidx, node_W, node_b, edge_table, convs, cls):
    raise NotImplementedError("write your pallas kernel here")


# speedup vs baseline: 1.9676x; 1.9676x over previous
import functools
import math

import jax
import jax.numpy as jnp
from jax import lax
from jax.experimental import pallas as pl
from jax.experimental.pallas import tpu as pltpu
from jax.experimental.pallas import tpu_sc as plsc

NV = 10000
NE = 5000
E = 320000
D = 128
ALPHA = 0.4
LAMDA = 0.5

NC = 2
NS = 16
NW = NC * NS
CH = 64
PWP = 10240
EP = NW * PWP
NCH = PWP // CH
NVP = 10240
NEP = 5120
RV = NVP // NS
RE = NEP // NS


@functools.cache
def _mesh():
    return plsc.VectorSubcoreMesh(
        core_axis_name="c", subcore_axis_name="s", num_cores=NC, num_subcores=NS
    )


def _f32(shape):
    return jax.ShapeDtypeStruct(shape, jnp.float32)


def _wid():
    return lax.axis_index("s") * NC + lax.axis_index("c")


def _sc_count(idx_s, ones128, z128, nrows):
    rps = nrows // NS

    @functools.partial(
        pl.kernel,
        out_type=_f32((NC, nrows, D)),
        mesh=_mesh(),
        scratch_types=[
            pltpu.VMEM((CH,), jnp.int32),
            pltpu.VMEM((CH,), jnp.int32),
            pltpu.VMEM((CH, D), jnp.float32),
            pltpu.VMEM_SHARED((nrows, D), jnp.float32),
            pltpu.SemaphoreType.DMA((2,)),
        ],
    )
    def k(ix_h, ones_h, z_h, out_h, x0, x1, onesv, acc, sem):
        c = lax.axis_index("c")
        s = lax.axis_index("s")
        wid = _wid()
        xb = (x0, x1)
        assert rps <= RV
        pltpu.sync_copy(ones_h, onesv)
        base_r = s * rps
        pltpu.sync_copy(z_h.at[pl.ds(0, rps)], acc.at[pl.ds(base_r, rps)])
        plsc.subcore_barrier()

        def issue_idx(j, b):
            pltpu.async_copy(ix_h.at[wid, j], xb[b], sem.at[b])

        def wait_idx(j, b):
            pltpu.make_async_copy(ix_h.at[wid, j], xb[b], sem.at[b]).wait()

        issue_idx(0, 0)
        issue_idx(1, 1)

        def two_chunks(g, carry):
            for b in range(2):
                j = g * 2 + b
                wait_idx(j, b)
                pltpu.sync_copy(onesv, acc.at[xb[b]], add=True)

                @pl.when(j + 2 < NCH)
                def _():
                    issue_idx(j + 2, b)
            return carry

        lax.fori_loop(0, NCH // 2, two_chunks, 0)
        plsc.subcore_barrier()
        pltpu.sync_copy(acc.at[pl.ds(base_r, rps)], out_h.at[c, pl.ds(base_r, rps)])

    return k(idx_s, ones128, z128)


def _sc_seg_sum(table, vidx_g, eidx_s, z128):
    @functools.partial(
        pl.kernel,
        out_type=_f32((NC, NEP, D)),
        mesh=_mesh(),
        scratch_types=[
            pltpu.VMEM((CH,), jnp.int32),
            pltpu.VMEM((CH,), jnp.int32),
            pltpu.VMEM((CH,), jnp.int32),
            pltpu.VMEM((CH,), jnp.int32),
            pltpu.VMEM((CH, D), jnp.float32),
            pltpu.VMEM((CH, D), jnp.float32),
            pltpu.VMEM_SHARED((NEP, D), jnp.float32),
            pltpu.SemaphoreType.DMA((6,)),
        ],
    )
    def k(tab_h, six_h, dix_h, z_h, out_h, sx0, sx1, dx0, dx1, rb0, rb1, acc, sem):
        c = lax.axis_index("c")
        s = lax.axis_index("s")
        wid = _wid()
        sxb = (sx0, sx1)
        dxb = (dx0, dx1)
        rb = (rb0, rb1)
        pltpu.sync_copy(z_h.at[pl.ds(0, RE)], acc.at[pl.ds(s * RE, RE)])
        plsc.subcore_barrier()

        def issue_idx(j, b):
            pltpu.async_copy(six_h.at[wid, j], sxb[b], sem.at[b])
            pltpu.async_copy(dix_h.at[wid, j], dxb[b], sem.at[2 + b])

        def wait_idx(j, b):
            pltpu.make_async_copy(six_h.at[wid, j], sxb[b], sem.at[b]).wait()
            pltpu.make_async_copy(dix_h.at[wid, j], dxb[b], sem.at[2 + b]).wait()

        def issue_gat(b):
            pltpu.async_copy(tab_h.at[sxb[b]], rb[b], sem.at[4 + b])

        def wait_gat(b):
            pltpu.make_async_copy(tab_h.at[sxb[b]], rb[b], sem.at[4 + b]).wait()

        issue_idx(0, 0)
        issue_idx(1, 1)
        wait_idx(0, 0)
        issue_gat(0)

        def two_chunks(g, carry):
            for b in range(2):
                j = g * 2 + b
                b1 = 1 - b
                wait_gat(b)

                @pl.when(j + 1 < NCH)
                def _():
                    wait_idx(j + 1, b1)
                    issue_gat(b1)

                pltpu.sync_copy(rb[b], acc.at[dxb[b]], add=True)

                @pl.when(j + 2 < NCH)
                def _():
                    issue_idx(j + 2, b)
            return carry

        lax.fori_loop(0, NCH // 2, two_chunks, 0)
        plsc.subcore_barrier()
        pltpu.sync_copy(acc.at[pl.ds(s * RE, RE)], out_h.at[c, pl.ds(s * RE, RE)])

    return k(table, vidx_g, eidx_s, z128)


def _sc_pair_gather(vtab, etab, vidx_g, eidx_g):
    @functools.partial(
        pl.kernel,
        out_type=(_f32((EP, D)), _f32((EP, D))),
        mesh=_mesh(),
        scratch_types=[
            pltpu.VMEM((CH,), jnp.int32),
            pltpu.VMEM((CH,), jnp.int32),
            pltpu.VMEM((CH,), jnp.int32),
            pltpu.VMEM((CH,), jnp.int32),
            pltpu.VMEM((CH, D), jnp.float32),
            pltpu.VMEM((CH, D), jnp.float32),
            pltpu.VMEM((CH, D), jnp.float32),
            pltpu.VMEM((CH, D), jnp.float32),
            pltpu.SemaphoreType.DMA((12,)),
        ],
    )
    def k(vtab_h, etab_h, vix_h, eix_h, uv_h, ue_h,
          vx0, vx1, ex0, ex1, vb0, vb1, eb0, eb1, sem):
        wid = _wid()
        vxb = (vx0, vx1)
        exb = (ex0, ex1)
        vb = (vb0, vb1)
        eb = (eb0, eb1)

        def issue_idx(j, b):
            pltpu.async_copy(vix_h.at[wid, j], vxb[b], sem.at[b])
            pltpu.async_copy(eix_h.at[wid, j], exb[b], sem.at[2 + b])

        def wait_idx(j, b):
            pltpu.make_async_copy(vix_h.at[wid, j], vxb[b], sem.at[b]).wait()
            pltpu.make_async_copy(eix_h.at[wid, j], exb[b], sem.at[2 + b]).wait()

        def issue_gat(b):
            pltpu.async_copy(vtab_h.at[vxb[b]], vb[b], sem.at[4 + b])
            pltpu.async_copy(etab_h.at[exb[b]], eb[b], sem.at[6 + b])

        def wait_gat(b):
            pltpu.make_async_copy(vtab_h.at[vxb[b]], vb[b], sem.at[4 + b]).wait()
            pltpu.make_async_copy(etab_h.at[exb[b]], eb[b], sem.at[6 + b]).wait()

        def issue_wr(j, b):
            base = wid * PWP + j * CH
            pltpu.async_copy(vb[b], uv_h.at[pl.ds(base, CH)], sem.at[8 + b])
            pltpu.async_copy(eb[b], ue_h.at[pl.ds(base, CH)], sem.at[10 + b])

        def wait_wr(j, b):
            base = wid * PWP + j * CH
            pltpu.make_async_copy(vb[b], uv_h.at[pl.ds(base, CH)], sem.at[8 + b]).wait()
            pltpu.make_async_copy(eb[b], ue_h.at[pl.ds(base, CH)], sem.at[10 + b]).wait()

        issue_idx(0, 0)
        issue_idx(1, 1)
        wait_idx(0, 0)
        issue_gat(0)

        def two_chunks(g, carry):
            for b in range(2):
                j = g * 2 + b
                b1 = 1 - b
                wait_gat(b)

                @pl.when(j + 1 < NCH)
                def _():
                    @pl.when(j >= 1)
                    def _():
                        wait_wr(j - 1, b1)

                    wait_idx(j + 1, b1)
                    issue_gat(b1)

                issue_wr(j, b)

                @pl.when(j + 2 < NCH)
                def _():
                    issue_idx(j + 2, b)
            return carry

        lax.fori_loop(0, NCH // 2, two_chunks, 0)
        wait_wr(NCH - 2, (NCH - 2) % 2)
        wait_wr(NCH - 1, (NCH - 1) % 2)

    return k(vtab, etab, vidx_g, eidx_g)


def _sc_scatter_back(ecs, ys, eidx_g, vidx_s, z128):
    @functools.partial(
        pl.kernel,
        out_type=_f32((NC, NVP, D)),
        mesh=_mesh(),
        scratch_types=[
            pltpu.VMEM((CH,), jnp.int32),
            pltpu.VMEM((CH,), jnp.int32),
            pltpu.VMEM((CH,), jnp.int32),
            pltpu.VMEM((CH,), jnp.int32),
            pltpu.VMEM((CH, D), jnp.float32),
            pltpu.VMEM((CH, D), jnp.float32),
            pltpu.VMEM((CH, D), jnp.float32),
            pltpu.VMEM((CH, D), jnp.float32),
            pltpu.VMEM_SHARED((NVP, D), jnp.float32),
            pltpu.SemaphoreType.DMA((8,)),
        ],
    )
    def k(ecs_h, ys_h, eix_h, vix_h, z_h, out_h,
          ex0, ex1, vx0, vx1, gb0, gb1, yb0, yb1, acc, sem):
        c = lax.axis_index("c")
        s = lax.axis_index("s")
        wid = _wid()
        exb = (ex0, ex1)
        vxb = (vx0, vx1)
        gb = (gb0, gb1)
        yb = (yb0, yb1)
        pltpu.sync_copy(z_h.at[pl.ds(0, RV)], acc.at[pl.ds(s * RV, RV)])
        plsc.subcore_barrier()

        def issue_idx(j, b):
            pltpu.async_copy(eix_h.at[wid, j], exb[b], sem.at[b])
            pltpu.async_copy(vix_h.at[wid, j], vxb[b], sem.at[2 + b])

        def wait_idx(j, b):
            pltpu.make_async_copy(eix_h.at[wid, j], exb[b], sem.at[b]).wait()
            pltpu.make_async_copy(vix_h.at[wid, j], vxb[b], sem.at[2 + b]).wait()

        def issue_ld(j, b):
            pltpu.async_copy(ecs_h.at[exb[b]], gb[b], sem.at[4 + b])
            base = wid * PWP + j * CH
            pltpu.async_copy(ys_h.at[pl.ds(base, CH)], yb[b], sem.at[6 + b])

        def wait_ld(j, b):
            pltpu.make_async_copy(ecs_h.at[exb[b]], gb[b], sem.at[4 + b]).wait()
            base = wid * PWP + j * CH
            pltpu.make_async_copy(ys_h.at[pl.ds(base, CH)], yb[b], sem.at[6 + b]).wait()

        issue_idx(0, 0)
        issue_idx(1, 1)
        wait_idx(0, 0)
        issue_ld(0, 0)

        def two_chunks(g, carry):
            for b in range(2):
                j = g * 2 + b
                b1 = 1 - b
                wait_ld(j, b)

                @pl.when(j + 1 < NCH)
                def _():
                    wait_idx(j + 1, b1)
                    issue_ld(j + 1, b1)

                pltpu.sync_copy(gb[b], acc.at[vxb[b]], add=True)
                pltpu.sync_copy(yb[b], acc.at[vxb[b]], add=True)

                @pl.when(j + 2 < NCH)
                def _():
                    issue_idx(j + 2, b)
            return carry

        lax.fori_loop(0, NCH // 2, two_chunks, 0)
        plsc.subcore_barrier()
        pltpu.sync_copy(acc.at[pl.ds(s * RV, RV)], out_h.at[c, pl.ds(s * RV, RV)])

    return k(ecs, ys, eidx_g, vidx_s, z128)


def _ln(x, g, b):
    mu = jnp.mean(x, axis=-1, keepdims=True)
    var = jnp.mean((x - mu) ** 2, axis=-1, keepdims=True)
    return (x - mu) / jnp.sqrt(var + 1e-5) * g + b


def _full(shape):
    return pl.BlockSpec(shape, lambda i: (0,) * len(shape))


def _rows(br, w):
    return pl.BlockSpec((br, w), lambda i: (i, 0))


def _mlp_block(x, p):
    if "ln_in_g" in p:
        x = _ln(x, p["ln_in_g"], p["ln_in_b"])
    h = jax.nn.relu(jnp.dot(x, p["W1"], preferred_element_type=jnp.float32) + p["b1"])
    h = _ln(h, p["ln_g"], p["ln_b"])
    return jnp.dot(h, p["W2"], preferred_element_type=jnp.float32) + p["b2"]


def _prep(p):
    out = {}
    for k, v in p.items():
        out[k] = v.reshape(1, -1) if v.ndim == 1 else v
    return out


def _mlp_param_specs(p):
    keys = sorted(p.keys())
    return keys, [_full(p[k].shape) for k in keys]


def _tc_mlp(x, p, br):
    p = _prep(p)
    n, din = x.shape
    dout = p["W2"].shape[1]
    keys, specs = _mlp_param_specs(p)

    def body(x_ref, *refs):
        pr = {k: r[...] for k, r in zip(keys, refs[:-1])}
        refs[-1][...] = _mlp_block(x_ref[...], pr)

    return pl.pallas_call(
        body,
        grid=(n // br,),
        in_specs=[_rows(br, din)] + specs,
        out_specs=_rows(br, dout),
        out_shape=_f32((n, dout)),
    )(x, *[p[k] for k in keys])


def _tc_prelude(v, node_W, node_b, n2m):
    n2m = _prep(n2m)
    keys, specs = _mlp_param_specs(n2m)
    br = 1000

    def body(v_ref, w_ref, b_ref, *refs):
        pr = {k: r[...] for k, r in zip(keys, refs[:-2])}
        v0 = jax.nn.relu(
            jnp.dot(v_ref[...], w_ref[...], preferred_element_type=jnp.float32)
            + b_ref[...]
        )
        refs[-2][...] = v0
        refs[-1][...] = _mlp_block(v0, pr)

    return pl.pallas_call(
        body,
        grid=(NV // br,),
        in_specs=[_rows(br, D), _full((D, D)), _full((1, D))] + specs,
        out_specs=(_rows(br, D), _rows(br, D)),
        out_shape=(_f32((NV, D)), _f32((NV, D))),
    )(v, node_W, node_b.reshape(1, D), *[n2m[k] for k in keys])


def _tc_relu(x, br):
    n, w = x.shape

    def body(x_ref, o_ref):
        o_ref[...] = jax.nn.relu(x_ref[...])

    return pl.pallas_call(
        body,
        grid=(n // br,),
        in_specs=[_rows(br, w)],
        out_specs=_rows(br, w),
        out_shape=_f32((n, w)),
    )(x)


def _tc_scale_counts(parts, br):
    np_ = parts.shape[1]

    def body(p_ref, o_ref):
        c = p_ref[0][:, :16] + p_ref[1][:, :16]
        o_ref[...] = (1.0 - ALPHA) / jnp.maximum(c, 1.0)

    return pl.pallas_call(
        body,
        grid=(np_ // br,),
        in_specs=[pl.BlockSpec((NC, br, D), lambda i: (0, i, 0))],
        out_specs=_rows(br, 16),
        out_shape=_f32((np_, 16)),
    )(parts)


def _tc_edge_stage(esum_parts, scale_e, e0, e_cur, beta):
    br = 1000

    def body(p_ref, sc_ref, e0_ref, ec_ref, edge_ref, ecs_ref, er_ref):
        es = p_ref[0] + p_ref[1]
        edge = es * sc_ref[:, :1] + ALPHA * e0_ref[...]
        edge_ref[...] = edge
        ecs_ref[...] = (1.0 - beta) * ec_ref[...]
        er_ref[...] = jax.nn.relu(edge)

    return pl.pallas_call(
        body,
        grid=(NE // br,),
        in_specs=[
            pl.BlockSpec((NC, br, D), lambda i: (0, i, 0)),
            _rows(br, 16),
            _rows(br, D),
            _rows(br, D),
        ],
        out_specs=(_rows(br, D), _rows(br, D), _rows(br, D)),
        out_shape=(_f32((NE, D)), _f32((NE, D)), _f32((NE, D))),
    )(esum_parts, scale_e, e0, e_cur)


def _tc_pair_mlp(uv, ue, e2m, beta):
    p = _prep(e2m)
    br = 4096
    gv, ge = p["ln_in_g"][:, :D], p["ln_in_g"][:, D:]
    bv, be = p["ln_in_b"][:, :D], p["ln_in_b"][:, D:]
    w1v, w1e = p["W1"][:D], p["W1"][D:]

    def body(uv_ref, ue_ref, gv_r, ge_r, bv_r, be_r, w1v_r, w1e_r, b1_r,
             lng_r, lnb_r, w2_r, b2_r, o_ref):
        xv = uv_ref[...]
        xe = ue_ref[...]
        s = (jnp.sum(xv, axis=-1, keepdims=True) + jnp.sum(xe, axis=-1, keepdims=True)) / (2 * D)
        q = (jnp.sum(xv * xv, axis=-1, keepdims=True) + jnp.sum(xe * xe, axis=-1, keepdims=True)) / (2 * D)
        r = lax.rsqrt(jnp.maximum(q - s * s, 0.0) + 1e-5)
        nv = (xv - s) * r * gv_r[...] + bv_r[...]
        ne_ = (xe - s) * r * ge_r[...] + be_r[...]
        h = jax.nn.relu(
            jnp.dot(nv, w1v_r[...], preferred_element_type=jnp.float32)
            + jnp.dot(ne_, w1e_r[...], preferred_element_type=jnp.float32)
            + b1_r[...]
        )
        h = _ln(h, lng_r[...], lnb_r[...])
        o_ref[...] = beta * (
            jnp.dot(h, w2_r[...], preferred_element_type=jnp.float32) + b2_r[...]
        )

    return pl.pallas_call(
        body,
        grid=(EP // br,),
        in_specs=[
            _rows(br, D), _rows(br, D),
            _full((1, D)), _full((1, D)), _full((1, D)), _full((1, D)),
            _full((D, D)), _full((D, D)), _full((1, D)),
            _full((1, D)), _full((1, D)), _full((D, D)), _full((1, D)),
        ],
        out_specs=_rows(br, D),
        out_shape=_f32((EP, D)),
    )(uv, ue, gv, ge, bv, be, w1v, w1e, p["b1"], p["ln_g"], p["ln_b"], p["W2"], p["b2"])


def _tc_node_stage(nsum_parts, scale_v, v0, agg, beta):
    agg = _prep(agg)
    keys, specs = _mlp_param_specs(agg)
    br = 1000

    def body(p_ref, sc_ref, v0_ref, *refs):
        pr = {k: r[...] for k, r in zip(keys, refs[:-1])}
        ns = p_ref[0] + p_ref[1]
        node = ns * sc_ref[:, :1] + ALPHA * v0_ref[...]
        m = _mlp_block(node, pr)
        refs[-1][...] = jax.nn.relu(beta * m + (1.0 - beta) * node)

    return pl.pallas_call(
        body,
        grid=(NV // br,),
        in_specs=[
            pl.BlockSpec((NC, br, D), lambda i: (0, i, 0)),
            _rows(br, 16),
            _rows(br, D),
        ] + specs,
        out_specs=_rows(br, D),
        out_shape=_f32((NV, D)),
    )(nsum_parts, scale_v, v0, *[agg[k] for k in keys])


def kernel(v, e, vidx, eidx, node_W, node_b, edge_table, convs, cls):
    del e
    npad = EP - E
    vidx_g = jnp.concatenate([vidx, jnp.zeros((npad,), jnp.int32)]).reshape(NW, NCH, CH)
    eidx_g = jnp.concatenate([eidx, jnp.zeros((npad,), jnp.int32)]).reshape(NW, NCH, CH)
    vidx_s = jnp.concatenate(
        [vidx, jnp.full((npad,), NVP - 1, jnp.int32)]).reshape(NW, NCH, CH)
    eidx_s = jnp.concatenate(
        [eidx, jnp.full((npad,), NEP - 1, jnp.int32)]).reshape(NW, NCH, CH)
    z128 = jnp.zeros((RV, D), jnp.float32)
    ones128 = jnp.ones((CH, D), jnp.float32)

    cv_parts = _sc_count(vidx_s, ones128, z128, NVP)
    ce_parts = _sc_count(eidx_s, ones128, z128, NEP)
    scale_v = _tc_scale_counts(cv_parts, 1280)
    scale_e = _tc_scale_counts(ce_parts, 1280)

    v0, node_msg = _tc_prelude(v, node_W, node_b, convs[0]["n2m"])
    e0 = _tc_relu(edge_table, 1000)
    v_cur, e_cur = v0, e0

    for i, cp in enumerate(convs):
        beta = math.log(LAMDA / (i + 1) + 1)
        if i > 0:
            node_msg = _tc_mlp(v_cur, cp["n2m"], 1000)
        esum_parts = _sc_seg_sum(node_msg, vidx_g, eidx_s, z128)
        edge, ecs, e_relu = _tc_edge_stage(esum_parts, scale_e, e0, e_cur, beta)
        uv, ue = _sc_pair_gather(v_cur, edge, vidx_g, eidx_g)
        ys = _tc_pair_mlp(uv, ue, cp["e2m"], beta)
        nsum_parts = _sc_scatter_back(ecs, ys, eidx_g, vidx_s, z128)
        v_cur = _tc_node_stage(nsum_parts, scale_v, v0, cp["agg"], beta)
        e_cur = e_relu

    pred = _tc_mlp(v_cur, cls, 1000)
    return (v_cur, e_cur, pred)
